# Initial kernel scaffold; baseline (speedup 1.0000x reference)
import jax
import jax.numpy as jnp
from jax.experimental import pallas as pl


def kernel(x_feat, x_angle, y_angle, W1, b1, W2, b2, W3, b3, W4, b4):
    raise NotImplementedError("write your pallas kernel here")


# speedup vs baseline: 2.1310x; 2.1310x over previous
import functools

import jax
import jax.numpy as jnp
from jax import lax
from jax.experimental import pallas as pl
from jax.experimental.pallas import tpu as pltpu

DIM = 1024
BLK = 512

_DN_TT = (((0,), (1,)), ((), ()))
_DN_NT = (((1,), (1,)), ((), ()))
_DN_NN = (((1,), (0,)), ((), ()))


def _combine_body(w1, w2, w3, w4, b1, b2, b3, b4, m2, m3, c2, c3):
    W1 = w1[...]
    W2 = w2[...]
    W3 = w3[...]
    W4 = w4[...]
    M2 = lax.dot_general(W3, W4, _DN_TT, preferred_element_type=jnp.float32)
    P = lax.dot_general(W1, W2, _DN_TT, preferred_element_type=jnp.float32)
    m2[...] = M2
    m3[...] = lax.dot_general(P, M2, _DN_NN, preferred_element_type=jnp.float32)
    c2v = lax.dot_general(b3[...], W4, _DN_NT,
                          preferred_element_type=jnp.float32) + b4[...]
    c2[...] = c2v
    t = lax.dot_general(b1[...], W2, _DN_NT,
                        preferred_element_type=jnp.float32) + b2[...]
    c3[...] = lax.dot_general(t, M2, _DN_NN,
                              preferred_element_type=jnp.float32) + c2v


def _apply_body(x, xa, ya, m2, m3, c2, c3, out):
    xb = x[...]
    y2 = lax.dot_general(xb, m2[...], _DN_NN,
                         preferred_element_type=jnp.float32) + c2[...]
    y3 = lax.dot_general(xb, m3[...], _DN_NN,
                         preferred_element_type=jnp.float32) + c3[...]

    def level(a):
        a0, a1, a2 = a[:, 0:1], a[:, 1:2], a[:, 2:3]
        return jnp.where((a0 >= a1) & (a0 >= a2), 0,
                         jnp.where(a1 >= a2, 1, 2))

    lvl = jnp.maximum(level(xa[...]), level(ya[...]))
    out[...] = jnp.where(lvl == 2, y3, jnp.where(lvl == 1, y2, xb))


def kernel(x_feat, x_angle, y_angle, W1, b1, W2, b2, W3, b3, W4, b4):
    b1r = b1.reshape(1, DIM)
    b2r = b2.reshape(1, DIM)
    b3r = b3.reshape(1, DIM)
    b4r = b4.reshape(1, DIM)

    mat = jax.ShapeDtypeStruct((DIM, DIM), jnp.float32)
    vec = jax.ShapeDtypeStruct((1, DIM), jnp.float32)
    M2, M3, c2, c3 = pl.pallas_call(
        _combine_body,
        out_shape=(mat, mat, vec, vec),
    )(W1, W2, W3, W4, b1r, b2r, b3r, b4r)

    n = x_feat.shape[0]
    grid = (n // BLK,)
    full = pl.BlockSpec((DIM, DIM), lambda i: (0, 0))
    row = pl.BlockSpec((1, DIM), lambda i: (0, 0))
    out = pl.pallas_call(
        _apply_body,
        grid=grid,
        in_specs=[
            pl.BlockSpec((BLK, DIM), lambda i: (i, 0)),
            pl.BlockSpec((BLK, 3), lambda i: (i, 0)),
            pl.BlockSpec((BLK, 3), lambda i: (i, 0)),
            full, full, row, row,
        ],
        out_specs=pl.BlockSpec((BLK, DIM), lambda i: (i, 0)),
        out_shape=jax.ShapeDtypeStruct((n, DIM), jnp.float32),
    )(x_feat, x_angle, y_angle, M2, M3, c2, c3)
    return out
